# baseline (speedup 1.0000x reference)
import jax
import jax.numpy as jnp
from jax.experimental import pallas as pl
from jax.experimental.pallas import tpu as pltpu

NUM_EXPERTS = 64
TOP_K = 2
HIDDEN = 1024
FFN = 512


def _routing_body(x_ref, gate_ref, w_ref, ids_ref, flags_ref):
    x = x_ref[...]
    gate = gate_ref[...]
    logits = jax.lax.dot_general(
        x, gate, (((1,), (1,)), ((), ())), preferred_element_type=jnp.float32
    )
    T, E = logits.shape
    e_iota = jax.lax.broadcasted_iota(jnp.int32, (T, E), 1)

    l1 = jnp.max(logits, axis=-1, keepdims=True)
    i1 = jnp.min(jnp.where(logits == l1, e_iota, E), axis=-1, keepdims=True)
    masked = jnp.where(e_iota == i1, -jnp.inf, logits)
    l2 = jnp.max(masked, axis=-1, keepdims=True)
    i2 = jnp.min(jnp.where(masked == l2, e_iota, E), axis=-1, keepdims=True)
    w1 = 1.0 / (1.0 + jnp.exp(l2 - l1))
    w2 = 1.0 - w1
    w_dense = (jnp.where(e_iota == i1, w1, 0.0)
               + jnp.where(e_iota == i2, w2, 0.0))
    w_ref[...] = w_dense
    active_row = jnp.sum((w_dense > 0.0).astype(jnp.int32),
                         axis=0, keepdims=True) > 0

    logits_t = jax.lax.dot_general(
        gate, x, (((1,), (1,)), ((), ())), preferred_element_type=jnp.float32
    )
    et_iota = jax.lax.broadcasted_iota(jnp.int32, (E, T), 0)
    l1c = jnp.max(logits_t, axis=0, keepdims=True)
    i1c = jnp.min(jnp.where(logits_t == l1c, et_iota, E), axis=0, keepdims=True)
    masked_c = jnp.where(et_iota == i1c, -jnp.inf, logits_t)
    l2c = jnp.max(masked_c, axis=0, keepdims=True)
    i2c = jnp.min(jnp.where(masked_c == l2c, et_iota, E), axis=0, keepdims=True)
    routed_t = (et_iota == i1c) | (et_iota == i2c)
    active_col = jnp.sum(routed_t.astype(jnp.int32),
                         axis=1, keepdims=True) > 0

    e_row = jax.lax.broadcasted_iota(jnp.int32, (1, E), 1)
    e_col = jax.lax.broadcasted_iota(jnp.int32, (E, 1), 0)
    key_row = jnp.where(active_row, e_row, e_row + E)
    key_col = jnp.where(active_col, e_col, e_col + E)
    rank_col = jnp.sum((key_col > key_row).astype(jnp.int32),
                       axis=1, keepdims=True)
    hit = (rank_col == e_row).astype(jnp.int32)
    perm = jnp.sum(hit * e_col, axis=0, keepdims=True)
    flags = jnp.sum(hit * active_col.astype(jnp.int32),
                    axis=0, keepdims=True)
    last_active = jnp.max(jnp.where(active_row, e_row, 0),
                          axis=1, keepdims=True)
    ids_ref[...] = jnp.where(flags > 0, perm, last_active)
    flags_ref[...] = flags


def _moe_body(ids_ref, flags_ref, x_ref, w_ref, wg_ref, wu_ref, wd_ref, out_ref):
    i = pl.program_id(0)

    @pl.when(i == 0)
    def _init():
        out_ref[...] = jnp.zeros_like(out_ref)

    @pl.when(flags_ref[i] > 0)
    def _step():
        x = x_ref[...]
        g = jax.lax.dot_general(
            x, wg_ref[0], (((1,), (1,)), ((), ())),
            preferred_element_type=jnp.float32)
        u = jax.lax.dot_general(
            x, wu_ref[0], (((1,), (1,)), ((), ())),
            preferred_element_type=jnp.float32)
        h = (g * jax.nn.sigmoid(g)) * u
        o = jax.lax.dot_general(
            h, wd_ref[0], (((1,), (1,)), ((), ())),
            preferred_element_type=jnp.float32)
        T, E = w_ref.shape
        e_iota = jax.lax.broadcasted_iota(jnp.int32, (T, E), 1)
        w_col = jnp.sum(
            jnp.where(e_iota == ids_ref[i], w_ref[...], 0.0),
            axis=-1, keepdims=True)
        out_ref[...] += o * w_col


def kernel(hidden_states, gate_w, Wg, Wu, Wd):
    B, S, D = hidden_states.shape
    T = B * S
    E = NUM_EXPERTS
    F = FFN
    x = hidden_states.reshape(T, D)

    w_dense, ids, flags = pl.pallas_call(
        _routing_body,
        out_shape=[
            jax.ShapeDtypeStruct((T, E), jnp.float32),
            jax.ShapeDtypeStruct((1, E), jnp.int32),
            jax.ShapeDtypeStruct((1, E), jnp.int32),
        ],
    )(x, gate_w)
    ids = jnp.arange(E, dtype=jnp.int32)
    flags = jnp.ones((E,), jnp.int32)

    out = pl.pallas_call(
        _moe_body,
        grid_spec=pltpu.PrefetchScalarGridSpec(
            num_scalar_prefetch=2,
            grid=(E,),
            in_specs=[
                pl.BlockSpec((T, D), lambda i, ids, flags: (0, 0)),
                pl.BlockSpec((T, E), lambda i, ids, flags: (0, 0)),
                pl.BlockSpec((1, F, D), lambda i, ids, flags: (ids[i], 0, 0)),
                pl.BlockSpec((1, F, D), lambda i, ids, flags: (ids[i], 0, 0)),
                pl.BlockSpec((1, D, F), lambda i, ids, flags: (ids[i], 0, 0)),
            ],
            out_specs=pl.BlockSpec((T, D), lambda i, ids, flags: (0, 0)),
        ),
        out_shape=jax.ShapeDtypeStruct((T, D), jnp.float32),
    )(ids, flags, x, w_dense, Wg, Wu, Wd)

    return out.reshape(B, S, D)


# speedup vs baseline: 1.6918x; 1.6918x over previous
import jax
import jax.numpy as jnp
from jax.experimental import pallas as pl
from jax.experimental.pallas import tpu as pltpu

NUM_EXPERTS = 64
TOP_K = 2
HIDDEN = 1024
FFN = 512


def _routing_body(x_ref, gate_ref, w_ref, ids_ref, flags_ref):
    x = x_ref[...]
    gate = gate_ref[...]
    logits = jax.lax.dot_general(
        x, gate, (((1,), (1,)), ((), ())), preferred_element_type=jnp.float32
    )
    T, E = logits.shape
    e_iota = jax.lax.broadcasted_iota(jnp.int32, (T, E), 1)

    l1 = jnp.max(logits, axis=-1, keepdims=True)
    i1 = jnp.min(jnp.where(logits == l1, e_iota, E), axis=-1, keepdims=True)
    masked = jnp.where(e_iota == i1, -jnp.inf, logits)
    l2 = jnp.max(masked, axis=-1, keepdims=True)
    i2 = jnp.min(jnp.where(masked == l2, e_iota, E), axis=-1, keepdims=True)
    w1 = 1.0 / (1.0 + jnp.exp(l2 - l1))
    w2 = 1.0 - w1
    w_dense = (jnp.where(e_iota == i1, w1, 0.0)
               + jnp.where(e_iota == i2, w2, 0.0))
    w_ref[...] = w_dense
    active_row = jnp.sum((w_dense > 0.0).astype(jnp.int32),
                         axis=0, keepdims=True) > 0

    logits_t = jax.lax.dot_general(
        gate, x, (((1,), (1,)), ((), ())), preferred_element_type=jnp.float32
    )
    et_iota = jax.lax.broadcasted_iota(jnp.int32, (E, T), 0)
    l1c = jnp.max(logits_t, axis=0, keepdims=True)
    i1c = jnp.min(jnp.where(logits_t == l1c, et_iota, E), axis=0, keepdims=True)
    masked_c = jnp.where(et_iota == i1c, -jnp.inf, logits_t)
    l2c = jnp.max(masked_c, axis=0, keepdims=True)
    i2c = jnp.min(jnp.where(masked_c == l2c, et_iota, E), axis=0, keepdims=True)
    routed_t = (et_iota == i1c) | (et_iota == i2c)
    active_col = jnp.sum(routed_t.astype(jnp.int32),
                         axis=1, keepdims=True) > 0

    e_row = jax.lax.broadcasted_iota(jnp.int32, (1, E), 1)
    e_col = jax.lax.broadcasted_iota(jnp.int32, (E, 1), 0)
    key_row = jnp.where(active_row, e_row, e_row + E)
    key_col = jnp.where(active_col, e_col, e_col + E)
    rank_col = jnp.sum((key_col > key_row).astype(jnp.int32),
                       axis=1, keepdims=True)
    hit = (rank_col == e_row).astype(jnp.int32)
    perm = jnp.sum(hit * e_col, axis=0, keepdims=True)
    flags = jnp.sum(hit * active_col.astype(jnp.int32),
                    axis=0, keepdims=True)
    n_act = jnp.sum(active_col.astype(jnp.int32), axis=0, keepdims=True)
    last1 = jnp.max(jnp.where(active_row, e_row, 0),
                    axis=1, keepdims=True)
    last2 = jnp.max(jnp.where(active_row & (e_row < last1), e_row, 0),
                    axis=1, keepdims=True)
    last2 = jnp.where(n_act > 1, last2, last1)
    pad_ids = jnp.where(((n_act - 1 - e_row) & 1) == 0, last1, last2)
    ids_ref[...] = jnp.where(flags > 0, perm, pad_ids)
    flags_ref[...] = flags


def _expert_contrib(x, w_ref, wg, wu, wd, eid):
    g = jax.lax.dot_general(
        x, wg, (((1,), (1,)), ((), ())), preferred_element_type=jnp.float32)
    u = jax.lax.dot_general(
        x, wu, (((1,), (1,)), ((), ())), preferred_element_type=jnp.float32)
    h = (g * jax.nn.sigmoid(g)) * u
    o = jax.lax.dot_general(
        h, wd, (((1,), (1,)), ((), ())), preferred_element_type=jnp.float32)
    T, E = w_ref.shape
    e_iota = jax.lax.broadcasted_iota(jnp.int32, (T, E), 1)
    w_col = jnp.sum(jnp.where(e_iota == eid, w_ref[...], 0.0),
                    axis=-1, keepdims=True)
    return o * w_col


def _moe_body(ids_ref, flags_ref, x_ref, w_ref,
              wg_a, wu_a, wd_a, wg_b, wu_b, wd_b, out_ref):
    j = pl.program_id(0)

    @pl.when(j == 0)
    def _init():
        out_ref[...] = jnp.zeros_like(out_ref)

    @pl.when(flags_ref[2 * j] > 0)
    def _slot_a():
        out_ref[...] += _expert_contrib(
            x_ref[...], w_ref, wg_a[0], wu_a[0], wd_a[0], ids_ref[2 * j])

    @pl.when(flags_ref[2 * j + 1] > 0)
    def _slot_b():
        out_ref[...] += _expert_contrib(
            x_ref[...], w_ref, wg_b[0], wu_b[0], wd_b[0], ids_ref[2 * j + 1])


def kernel(hidden_states, gate_w, Wg, Wu, Wd):
    B, S, D = hidden_states.shape
    T = B * S
    E = NUM_EXPERTS
    F = FFN
    x = hidden_states.reshape(T, D)

    w_dense, ids, flags = pl.pallas_call(
        _routing_body,
        out_shape=[
            jax.ShapeDtypeStruct((T, E), jnp.float32),
            jax.ShapeDtypeStruct((1, E), jnp.int32),
            jax.ShapeDtypeStruct((1, E), jnp.int32),
        ],
    )(x, gate_w)
    ids = ids.reshape(E)
    flags = flags.reshape(E)

    out = pl.pallas_call(
        _moe_body,
        grid_spec=pltpu.PrefetchScalarGridSpec(
            num_scalar_prefetch=2,
            grid=(E // 2,),
            in_specs=[
                pl.BlockSpec((T, D), lambda j, ids, flags: (0, 0)),
                pl.BlockSpec((T, E), lambda j, ids, flags: (0, 0)),
                pl.BlockSpec((1, F, D), lambda j, ids, flags: (ids[2 * j], 0, 0)),
                pl.BlockSpec((1, F, D), lambda j, ids, flags: (ids[2 * j], 0, 0)),
                pl.BlockSpec((1, D, F), lambda j, ids, flags: (ids[2 * j], 0, 0)),
                pl.BlockSpec((1, F, D), lambda j, ids, flags: (ids[2 * j + 1], 0, 0)),
                pl.BlockSpec((1, F, D), lambda j, ids, flags: (ids[2 * j + 1], 0, 0)),
                pl.BlockSpec((1, D, F), lambda j, ids, flags: (ids[2 * j + 1], 0, 0)),
            ],
            out_specs=pl.BlockSpec((T, D), lambda j, ids, flags: (0, 0)),
        ),
        out_shape=jax.ShapeDtypeStruct((T, D), jnp.float32),
    )(ids, flags, x, w_dense, Wg, Wu, Wd, Wg, Wu, Wd)

    return out.reshape(B, S, D)


# speedup vs baseline: 1.7221x; 1.0179x over previous
import jax
import jax.numpy as jnp
from jax.experimental import pallas as pl
from jax.experimental.pallas import tpu as pltpu

NUM_EXPERTS = 64
TOP_K = 2
HIDDEN = 1024
FFN = 512
NW = 4


def _routing_body(x_ref, gate_ref, w_ref, ids_ref, flags_ref):
    x = x_ref[...]
    gate = gate_ref[...]
    logits = jax.lax.dot_general(
        x, gate, (((1,), (1,)), ((), ())), preferred_element_type=jnp.float32
    )
    T, E = logits.shape
    e_iota = jax.lax.broadcasted_iota(jnp.int32, (T, E), 1)

    l1 = jnp.max(logits, axis=-1, keepdims=True)
    i1 = jnp.min(jnp.where(logits == l1, e_iota, E), axis=-1, keepdims=True)
    masked = jnp.where(e_iota == i1, -jnp.inf, logits)
    l2 = jnp.max(masked, axis=-1, keepdims=True)
    i2 = jnp.min(jnp.where(masked == l2, e_iota, E), axis=-1, keepdims=True)
    w1 = 1.0 / (1.0 + jnp.exp(l2 - l1))
    w2 = 1.0 - w1
    w_dense = (jnp.where(e_iota == i1, w1, 0.0)
               + jnp.where(e_iota == i2, w2, 0.0))
    w_ref[...] = w_dense
    active_row = jnp.sum((w_dense > 0.0).astype(jnp.int32),
                         axis=0, keepdims=True) > 0

    logits_t = jax.lax.dot_general(
        gate, x, (((1,), (1,)), ((), ())), preferred_element_type=jnp.float32
    )
    et_iota = jax.lax.broadcasted_iota(jnp.int32, (E, T), 0)
    l1c = jnp.max(logits_t, axis=0, keepdims=True)
    i1c = jnp.min(jnp.where(logits_t == l1c, et_iota, E), axis=0, keepdims=True)
    masked_c = jnp.where(et_iota == i1c, -jnp.inf, logits_t)
    l2c = jnp.max(masked_c, axis=0, keepdims=True)
    i2c = jnp.min(jnp.where(masked_c == l2c, et_iota, E), axis=0, keepdims=True)
    routed_t = (et_iota == i1c) | (et_iota == i2c)
    active_col = jnp.sum(routed_t.astype(jnp.int32),
                         axis=1, keepdims=True) > 0

    e_row = jax.lax.broadcasted_iota(jnp.int32, (1, E), 1)
    e_col = jax.lax.broadcasted_iota(jnp.int32, (E, 1), 0)
    key_row = jnp.where(active_row, e_row, e_row + E)
    key_col = jnp.where(active_col, e_col, e_col + E)
    rank_col = jnp.sum((key_col > key_row).astype(jnp.int32),
                       axis=1, keepdims=True)
    hit = (rank_col == e_row).astype(jnp.int32)
    perm = jnp.sum(hit * e_col, axis=0, keepdims=True)
    flags = jnp.sum(hit * active_col.astype(jnp.int32),
                    axis=0, keepdims=True)
    n_act = jnp.sum(active_col.astype(jnp.int32), axis=0, keepdims=True)
    lastk = []
    prev = jnp.max(jnp.where(active_row, e_row, 0), axis=1, keepdims=True)
    lastk.append(prev)
    for k in range(1, NW):
        cand = jnp.max(jnp.where(active_row & (e_row < prev), e_row, 0),
                       axis=1, keepdims=True)
        cand = jnp.where(n_act > k, cand, prev)
        lastk.append(cand)
        prev = cand
    back = (n_act - 1 - e_row) & (NW - 1)
    pad_ids = lastk[0]
    for k in range(1, NW):
        pad_ids = jnp.where(back == k, lastk[k], pad_ids)
    ids_ref[...] = jnp.where(flags > 0, perm, pad_ids)
    flags_ref[...] = flags


def _expert_contrib(x, w_ref, wg, wu, wd, eid):
    g = jax.lax.dot_general(
        x, wg, (((1,), (1,)), ((), ())), preferred_element_type=jnp.float32)
    u = jax.lax.dot_general(
        x, wu, (((1,), (1,)), ((), ())), preferred_element_type=jnp.float32)
    h = (g * jax.nn.sigmoid(g)) * u
    o = jax.lax.dot_general(
        h, wd, (((1,), (1,)), ((), ())), preferred_element_type=jnp.float32)
    T, E = w_ref.shape
    e_iota = jax.lax.broadcasted_iota(jnp.int32, (T, E), 1)
    w_col = jnp.sum(jnp.where(e_iota == eid, w_ref[...], 0.0),
                    axis=-1, keepdims=True)
    return o * w_col


def _moe_body(ids_ref, flags_ref, x_ref, w_ref, *refs):
    wrefs, out_ref = refs[:-1], refs[-1]
    j = pl.program_id(0)

    @pl.when(j == 0)
    def _init():
        out_ref[...] = jnp.zeros_like(out_ref)

    for k in range(NW):
        wg, wu, wd = wrefs[3 * k:3 * k + 3]

        @pl.when(flags_ref[NW * j + k] > 0)
        def _slot(wg=wg, wu=wu, wd=wd, k=k):
            out_ref[...] += _expert_contrib(
                x_ref[...], w_ref, wg[0], wu[0], wd[0], ids_ref[NW * j + k])


def kernel(hidden_states, gate_w, Wg, Wu, Wd):
    B, S, D = hidden_states.shape
    T = B * S
    E = NUM_EXPERTS
    F = FFN
    x = hidden_states.reshape(T, D)

    w_dense, ids, flags = pl.pallas_call(
        _routing_body,
        out_shape=[
            jax.ShapeDtypeStruct((T, E), jnp.float32),
            jax.ShapeDtypeStruct((1, E), jnp.int32),
            jax.ShapeDtypeStruct((1, E), jnp.int32),
        ],
    )(x, gate_w)
    ids = ids.reshape(E)
    flags = flags.reshape(E)

    def _wspec(k, shape):
        return pl.BlockSpec(
            (1,) + shape, lambda j, ids, flags, k=k: (ids[NW * j + k], 0, 0))

    weight_specs = []
    weight_args = []
    for k in range(NW):
        weight_specs += [_wspec(k, (F, D)), _wspec(k, (F, D)), _wspec(k, (D, F))]
        weight_args += [Wg, Wu, Wd]

    out = pl.pallas_call(
        _moe_body,
        grid_spec=pltpu.PrefetchScalarGridSpec(
            num_scalar_prefetch=2,
            grid=(E // NW,),
            in_specs=[
                pl.BlockSpec((T, D), lambda j, ids, flags: (0, 0)),
                pl.BlockSpec((T, E), lambda j, ids, flags: (0, 0)),
            ] + weight_specs,
            out_specs=pl.BlockSpec((T, D), lambda j, ids, flags: (0, 0)),
        ),
        out_shape=jax.ShapeDtypeStruct((T, D), jnp.float32),
    )(ids, flags, x, w_dense, *weight_args)

    return out.reshape(B, S, D)
